# Initial kernel scaffold; baseline (speedup 1.0000x reference)
import jax
import jax.numpy as jnp
from jax.experimental import pallas as pl


def kernel(x_L_CC, x_L_MLO, x_R_CC, x_R_MLO):
    raise NotImplementedError("write your pallas kernel here")


# speedup vs baseline: 1.1457x; 1.1457x over previous
import functools

import jax
import jax.numpy as jnp
from jax.experimental import pallas as pl
from jax.experimental.pallas import tpu as pltpu

_ROW_TILE = 256


def _pool4_kernel(a_ref, b_ref, c_ref, d_ref,
                  oa_ref, ob_ref, oc_ref, od_ref, *, hw_total, inv_hw):
    for x_ref, o_ref in ((a_ref, oa_ref), (b_ref, ob_ref),
                         (c_ref, oc_ref), (d_ref, od_ref)):
        x = x_ref[...].astype(jnp.float32)
        if hw_total % 128 != 0:
            col = jax.lax.broadcasted_iota(jnp.int32, x.shape, 1)
            x = jnp.where(col < hw_total, x, 0.0)
        s = jnp.sum(x, axis=-1, keepdims=True)
        o_ref[...] = (s * inv_hw).astype(o_ref.dtype)


def kernel(x_L_CC, x_L_MLO, x_R_CC, x_R_MLO):
    views = (x_L_CC, x_L_MLO, x_R_CC, x_R_MLO)
    n, c, h, w = views[0].shape
    nc, hw = n * c, h * w
    dtype = views[0].dtype

    flat = [v.reshape(nc, hw) for v in views]

    row_tile = nc if nc < _ROW_TILE else _ROW_TILE
    grid = (pl.cdiv(nc, row_tile),)

    kernel_fn = functools.partial(
        _pool4_kernel, hw_total=hw, inv_hw=float(1.0 / hw))

    itemsize = jnp.dtype(dtype).itemsize
    in_spec = pl.BlockSpec((row_tile, hw), lambda r: (r, 0))
    out_spec = pl.BlockSpec((row_tile, 1), lambda r: (r, 0))
    outs = pl.pallas_call(
        kernel_fn,
        out_shape=[jax.ShapeDtypeStruct((nc, 1), dtype)] * 4,
        grid=grid,
        in_specs=[in_spec] * 4,
        out_specs=[out_spec] * 4,
        compiler_params=pltpu.CompilerParams(
            dimension_semantics=("parallel",),
        ),
        cost_estimate=pl.CostEstimate(
            flops=4 * nc * hw,
            transcendentals=0,
            bytes_accessed=4 * (nc * hw + nc) * itemsize,
        ),
    )(*flat)

    names = ("L-CC", "L-MLO", "R-CC", "R-MLO")
    return {name: o.reshape(n, c) for name, o in zip(names, outs)}


# speedup vs baseline: 1.3140x; 1.1469x over previous
import functools

import jax
import jax.numpy as jnp
from jax.experimental import pallas as pl
from jax.experimental.pallas import tpu as pltpu


def _pool4_kernel(a_ref, b_ref, c_ref, d_ref,
                  oa_ref, ob_ref, oc_ref, od_ref, *, inv_hw):
    i = pl.program_id(1)
    for x_ref, o_ref in ((a_ref, oa_ref), (b_ref, ob_ref),
                         (c_ref, oc_ref), (d_ref, od_ref)):
        x = x_ref[0].astype(jnp.float32)
        s = jnp.sum(x, axis=(1, 2)) * inv_hw
        o_ref[pl.ds(i, 1), :] = s.astype(o_ref.dtype)[None, :]


def kernel(x_L_CC, x_L_MLO, x_R_CC, x_R_MLO):
    views = (x_L_CC, x_L_MLO, x_R_CC, x_R_MLO)
    n, c, h, w = views[0].shape
    dtype = views[0].dtype

    c_split = c // 128 if c % 128 == 0 else 1
    c_blk = c // c_split
    grid = (c_split, n)

    kernel_fn = functools.partial(_pool4_kernel, inv_hw=float(1.0 / (h * w)))

    in_spec = pl.BlockSpec((1, c_blk, h, w), lambda j, i: (i, j, 0, 0))
    out_spec = pl.BlockSpec((n, c_blk), lambda j, i: (0, j))
    itemsize = jnp.dtype(dtype).itemsize
    outs = pl.pallas_call(
        kernel_fn,
        out_shape=[jax.ShapeDtypeStruct((n, c), dtype)] * 4,
        grid=grid,
        in_specs=[in_spec] * 4,
        out_specs=[out_spec] * 4,
        compiler_params=pltpu.CompilerParams(
            dimension_semantics=("parallel", "arbitrary"),
        ),
        cost_estimate=pl.CostEstimate(
            flops=4 * n * c * h * w,
            transcendentals=0,
            bytes_accessed=4 * (n * c * h * w + n * c) * itemsize,
        ),
    )(*views)

    names = ("L-CC", "L-MLO", "R-CC", "R-MLO")
    return dict(zip(names, outs))


# speedup vs baseline: 10.8240x; 8.2374x over previous
import functools

import jax
import jax.numpy as jnp
from jax.experimental import pallas as pl
from jax.experimental.pallas import tpu as pltpu


def _pool4_kernel(a_ref, b_ref, c_ref, d_ref,
                  oa_ref, ob_ref, oc_ref, od_ref, *, inv_hw):
    i = pl.program_id(1)
    for x_ref, o_ref in ((a_ref, oa_ref), (b_ref, ob_ref),
                         (c_ref, oc_ref), (d_ref, od_ref)):
        x = x_ref[0].astype(jnp.float32)
        s = jnp.sum(x, axis=(0, 1)) * inv_hw
        o_ref[pl.ds(i, 1), :] = s.astype(o_ref.dtype)[None, :]


def kernel(x_L_CC, x_L_MLO, x_R_CC, x_R_MLO):
    views = (x_L_CC, x_L_MLO, x_R_CC, x_R_MLO)
    n, c, h, w = views[0].shape
    dtype = views[0].dtype

    nhwc = [jnp.transpose(v, (0, 2, 3, 1)) for v in views]

    c_split = 2 if c % 256 == 0 else 1
    c_blk = c // c_split
    grid = (c_split, n)

    kernel_fn = functools.partial(_pool4_kernel, inv_hw=float(1.0 / (h * w)))

    in_spec = pl.BlockSpec((1, h, w, c_blk), lambda j, i: (i, 0, 0, j))
    out_spec = pl.BlockSpec((n, c_blk), lambda j, i: (0, j))
    itemsize = jnp.dtype(dtype).itemsize
    outs = pl.pallas_call(
        kernel_fn,
        out_shape=[jax.ShapeDtypeStruct((n, c), dtype)] * 4,
        grid=grid,
        in_specs=[in_spec] * 4,
        out_specs=[out_spec] * 4,
        compiler_params=pltpu.CompilerParams(
            dimension_semantics=("parallel", "arbitrary"),
        ),
        cost_estimate=pl.CostEstimate(
            flops=4 * n * c * h * w,
            transcendentals=0,
            bytes_accessed=4 * (n * c * h * w + n * c) * itemsize,
        ),
    )(*nhwc)

    names = ("L-CC", "L-MLO", "R-CC", "R-MLO")
    return dict(zip(names, outs))
